# baseline (speedup 1.0000x reference)
import functools

import jax
import jax.numpy as jnp
from jax import lax
from jax.experimental import pallas as pl
from jax.experimental.pallas import tpu as pltpu
from jax.experimental.pallas import tpu_sc as plsc

N = 10000
D = 128
E = 320000
NEG_SLOPE = 0.01

NC = 2
NS = 16
L = 16
NW = NC * NS
E_PER_W = E // NW
CHUNK = 80
NCHUNK = E_PER_W // CHUNK

ROWS_PER_TILE = 624
TAIL_ROW0 = NS * ROWS_PER_TILE
TAIL = N - TAIL_ROW0

BM = 400

_DNUMS = lax.GatherDimensionNumbers(
    offset_dims=(), collapsed_slice_dims=(0,), start_index_map=(0,))


def _bcast_lane(vec16, t):
    return lax.gather(vec16, jnp.full((L, 1), t, jnp.int32), _DNUMS, (1,),
                      mode=lax.GatherScatterMode.PROMISE_IN_BOUNDS)


def _sc_spmm_body(rows_hbm, cols_hbm, vals_hbm, pre_hbm, out_hbm,
                  cidx0, cidx1, cidx2, ridx0, ridx1, ridx2,
                  sidx0, sidx1, sidx2, vals0, vals1, vals2,
                  buf0, buf1, buf2, acc_sh,
                  semg0, semg1, semg2, semi0, semi1, semi2,
                  sems0, sems1, sems2):
    cid = lax.axis_index("c")
    sid = lax.axis_index("s")
    wid = sid * NC + cid
    cidx = (cidx0, cidx1, cidx2)
    ridx = (ridx0, ridx1, ridx2)
    sidx = (sidx0, sidx1, sidx2)
    vals = (vals0, vals1, vals2)
    bufs = (buf0, buf1, buf2)
    semg = (semg0, semg1, semg2)
    semi = (semi0, semi1, semi2)
    sems = (sems0, sems1, sems2)
    base = wid * E_PER_W

    def idx_copies(c, b):
        off = base + c * CHUNK
        return (
            (cols_hbm.at[pl.ds(off, CHUNK)], cidx[b]),
            (rows_hbm.at[pl.ds(off, CHUNK)], ridx[b]),
            (vals_hbm.at[pl.ds(off, CHUNK)], vals[b]),
        )

    for src, dst in idx_copies(0, 0):
        pltpu.sync_copy(src, dst)
    pltpu.async_copy(pre_hbm.at[cidx[0]], bufs[0], semg[0])
    for src, dst in idx_copies(1, 1):
        pltpu.async_copy(src, dst, semi[1])
    for src, dst in idx_copies(2, 2):
        pltpu.async_copy(src, dst, semi[2])
    for src, dst in idx_copies(1, 1):
        pltpu.make_async_copy(src, dst, semi[1]).wait()
    pltpu.async_copy(pre_hbm.at[cidx[1]], bufs[1], semg[1])

    zero16 = jnp.zeros((L,), jnp.float32)

    def zrow(i, _):
        for j in range(D // L):
            buf2[i, pl.ds(j * L, L)] = zero16
        return ()

    lax.fori_loop(0, CHUNK, zrow, ())
    arow0 = sid * ROWS_PER_TILE
    for z in range(ROWS_PER_TILE // CHUNK):
        pltpu.sync_copy(buf2, acc_sh.at[pl.ds(arow0 + z * CHUNK, CHUNK), :])
    pltpu.sync_copy(buf2.at[pl.ds(0, 64), :],
                    acc_sh.at[pl.ds(arow0 + 560, 64), :])

    @pl.when(sid == 0)
    def _zero_tail():
        pltpu.sync_copy(buf2.at[pl.ds(0, TAIL), :],
                        acc_sh.at[pl.ds(TAIL_ROW0, TAIL), :])

    plsc.subcore_barrier()

    def slot(c, b):
        nb = (b + 2) % 3
        buf = bufs[b]
        @pl.when(c + 2 < NCHUNK)
        def _next_gather():
            for src, dst in idx_copies(c + 2, nb):
                pltpu.make_async_copy(src, dst, semi[nb]).wait()

            @pl.when(c >= 1)
            def _drain_prev_scatter():
                pltpu.make_async_copy(pre_hbm.at[pl.ds(0, CHUNK), :],
                                      bufs[nb], sems[nb]).wait()

            pltpu.async_copy(pre_hbm.at[cidx[nb]], bufs[nb], semg[nb])

        pltpu.make_async_copy(pre_hbm.at[cidx[b]], buf, semg[b]).wait()

        for q in range(CHUNK // L):
            vals16 = vals[b][pl.ds(q * L, L)]
            for t in range(L):
                e = q * L + t
                bval = _bcast_lane(vals16, t)
                for j in range(D // L):
                    sl = pl.ds(j * L, L)
                    buf[e, sl] = buf[e, sl] * bval

        for q in range(CHUNK // L):
            sl = pl.ds(q * L, L)
            sidx[b][sl] = ridx[b][sl]

        pltpu.async_copy(buf, acc_sh.at[sidx[b]], sems[b], add=True)

        @pl.when(c + 3 < NCHUNK)
        def _prefetch_idx():
            for src, dst in idx_copies(c + 3, b):
                pltpu.async_copy(src, dst, semi[b])

    def group(g, _):
        slot(3 * g, 0)
        slot(3 * g + 1, 1)
        slot(3 * g + 2, 2)
        return ()

    lax.fori_loop(0, NCHUNK // 3, group, ())
    slot(jnp.int32(NCHUNK - 2), 0)
    slot(jnp.int32(NCHUNK - 1), 1)

    for b in range(3):
        pltpu.make_async_copy(pre_hbm.at[pl.ds(0, CHUNK), :],
                              bufs[b], sems[b]).wait()
    plsc.subcore_barrier()

    pltpu.sync_copy(acc_sh.at[pl.ds(arow0, ROWS_PER_TILE), :],
                    out_hbm.at[cid, pl.ds(arow0, ROWS_PER_TILE), :])

    @pl.when(sid == 0)
    def _copy_tail():
        pltpu.sync_copy(acc_sh.at[pl.ds(TAIL_ROW0, TAIL), :],
                        out_hbm.at[cid, pl.ds(TAIL_ROW0, TAIL), :])


@jax.jit
def _sc_spmm(rows, cols, vals, pre):
    mesh = plsc.VectorSubcoreMesh(core_axis_name="c", subcore_axis_name="s",
                                  num_cores=NC, num_subcores=NS)
    return pl.kernel(
        _sc_spmm_body,
        out_type=jax.ShapeDtypeStruct((NC, N, D), jnp.float32),
        mesh=mesh,
        scratch_types=(
            [pltpu.VMEM((CHUNK,), jnp.int32)] * 9
            + [pltpu.VMEM((CHUNK,), jnp.float32)] * 3
            + [pltpu.VMEM((CHUNK, D), jnp.float32)] * 3
            + [pltpu.VMEM_SHARED((N, D), jnp.float32)]
            + [pltpu.SemaphoreType.DMA] * 9
        ),
    )(rows, cols, vals, pre)


def _tc_matmul_body(a_ref, pre_all_ref, agg_ref):
    agg_ref[...] = jnp.dot(a_ref[...], pre_all_ref[...],
                           preferred_element_type=jnp.float32)


@jax.jit
def _tc_matmul(A, pre):
    return pl.pallas_call(
        _tc_matmul_body,
        grid=(N // BM,),
        in_specs=[
            pl.BlockSpec((BM, N), lambda i: (i, 0)),
            pl.BlockSpec((N, D), lambda i: (0, 0)),
        ],
        out_specs=pl.BlockSpec((BM, D), lambda i: (i, 0)),
        out_shape=jax.ShapeDtypeStruct((N, D), jnp.float32),
    )(A, pre)


def _tc_epilogue_body(pre_ref, agg_ref, part_ref, w2_ref, b2_ref,
                      w3_ref, b3_ref, emb_ref, norm_ref):
    pre_i = pre_ref[...]
    all_emb = pre_i + agg_ref[...]
    sub_emb = pre_i * (part_ref[0] + part_ref[1])
    lin2 = jnp.dot(all_emb, w2_ref[...],
                   preferred_element_type=jnp.float32) + b2_ref[...]
    lin3 = jnp.dot(sub_emb, w3_ref[...],
                   preferred_element_type=jnp.float32) + b3_ref[...]
    emb = jax.nn.sigmoid(lin2) + jnp.where(lin3 > 0, lin3, NEG_SLOPE * lin3)
    emb_ref[...] = emb
    nrm = jnp.sqrt(jnp.sum(emb * emb, axis=1, keepdims=True))
    norm_ref[...] = emb / jnp.maximum(nrm, 1e-12)


@jax.jit
def _tc_epilogue(pre, agg, parts, W2, b2, W3, b3):
    return pl.pallas_call(
        _tc_epilogue_body,
        grid=(N // BM,),
        in_specs=[
            pl.BlockSpec((BM, D), lambda i: (i, 0)),
            pl.BlockSpec((BM, D), lambda i: (i, 0)),
            pl.BlockSpec((NC, BM, D), lambda i: (0, i, 0)),
            pl.BlockSpec((D, D), lambda i: (0, 0)),
            pl.BlockSpec((1, D), lambda i: (0, 0)),
            pl.BlockSpec((D, D), lambda i: (0, 0)),
            pl.BlockSpec((1, D), lambda i: (0, 0)),
        ],
        out_specs=[
            pl.BlockSpec((BM, D), lambda i: (i, 0)),
            pl.BlockSpec((BM, D), lambda i: (i, 0)),
        ],
        out_shape=[
            jax.ShapeDtypeStruct((N, D), jnp.float32),
            jax.ShapeDtypeStruct((N, D), jnp.float32),
        ],
    )(pre, agg, parts, W2, b2.reshape(1, D), W3, b3.reshape(1, D))


def kernel(A, sub_indices, sub_values, X,
           W2_0, b2_0, W3_0, b3_0,
           W2_1, b2_1, W3_1, b3_1):
    rows = sub_indices[0].astype(jnp.int32)
    cols = sub_indices[1].astype(jnp.int32)
    layer_params = [(W2_0, b2_0, W3_0, b3_0), (W2_1, b2_1, W3_1, b3_1)]
    pre = X
    finals = [2.0 * X]
    for (W2, b2, W3, b3) in layer_params:
        parts = _sc_spmm(rows, cols, sub_values, pre)
        agg = _tc_matmul(A, pre)
        emb, norm = _tc_epilogue(pre, agg, parts, W2, b2, W3, b3)
        pre = emb
        finals.append(norm)
    return jnp.concatenate(finals, axis=1)


# speedup vs baseline: 1.1788x; 1.1788x over previous
import functools

import jax
import jax.numpy as jnp
from jax import lax
from jax.experimental import pallas as pl
from jax.experimental.pallas import tpu as pltpu
from jax.experimental.pallas import tpu_sc as plsc

N = 10000
D = 128
E = 320000
NEG_SLOPE = 0.01

NC = 2
NS = 16
L = 16
NW = NC * NS
E_PER_W = E // NW
CHUNK = 80
NCHUNK = E_PER_W // CHUNK

ROWS_PER_TILE = 624
TAIL_ROW0 = NS * ROWS_PER_TILE
TAIL = N - TAIL_ROW0

BM = 400

_DNUMS = lax.GatherDimensionNumbers(
    offset_dims=(), collapsed_slice_dims=(0,), start_index_map=(0,))


def _bcast_lane(vec16, t):
    return lax.gather(vec16, jnp.full((L, 1), t, jnp.int32), _DNUMS, (1,),
                      mode=lax.GatherScatterMode.PROMISE_IN_BOUNDS)


def _sc_spmm_body(rows_hbm, cols_hbm, vals_hbm, pre_hbm, out_hbm,
                  cidx0, cidx1, ridx0, ridx1, sidx0, sidx1, vals0, vals1,
                  buf0, buf1, acc_sh, semg0, semg1, semi0, semi1,
                  sems0, sems1):
    cid = lax.axis_index("c")
    sid = lax.axis_index("s")
    wid = sid * NC + cid
    cidx = (cidx0, cidx1)
    ridx = (ridx0, ridx1)
    sidx = (sidx0, sidx1)
    vals = (vals0, vals1)
    bufs = (buf0, buf1)
    semg = (semg0, semg1)
    semi = (semi0, semi1)
    sems = (sems0, sems1)
    base = wid * E_PER_W

    def idx_copies(c, b):
        off = base + c * CHUNK
        return (
            (cols_hbm.at[pl.ds(off, CHUNK)], cidx[b]),
            (rows_hbm.at[pl.ds(off, CHUNK)], ridx[b]),
            (vals_hbm.at[pl.ds(off, CHUNK)], vals[b]),
        )

    for src, dst in idx_copies(0, 0):
        pltpu.sync_copy(src, dst)
    pltpu.async_copy(pre_hbm.at[cidx[0]], bufs[0], semg[0])
    for src, dst in idx_copies(1, 1):
        pltpu.async_copy(src, dst, semi[1])

    zero16 = jnp.zeros((L,), jnp.float32)

    def zrow(i, _):
        for j in range(D // L):
            buf1[i, pl.ds(j * L, L)] = zero16
        return ()

    lax.fori_loop(0, CHUNK, zrow, ())
    arow0 = sid * ROWS_PER_TILE
    for z in range(ROWS_PER_TILE // CHUNK):
        pltpu.sync_copy(buf1, acc_sh.at[pl.ds(arow0 + z * CHUNK, CHUNK), :])
    pltpu.sync_copy(buf1.at[pl.ds(0, 64), :],
                    acc_sh.at[pl.ds(arow0 + 560, 64), :])

    @pl.when(sid == 0)
    def _zero_tail():
        pltpu.sync_copy(buf1.at[pl.ds(0, TAIL), :],
                        acc_sh.at[pl.ds(TAIL_ROW0, TAIL), :])

    plsc.subcore_barrier()

    def slot(c, b):
        ob = 1 - b
        buf = bufs[b]
        @pl.when(c + 1 < NCHUNK)
        def _next_gather():
            for src, dst in idx_copies(c + 1, ob):
                pltpu.make_async_copy(src, dst, semi[ob]).wait()

            @pl.when(c >= 1)
            def _drain_prev_scatter():
                pltpu.make_async_copy(pre_hbm.at[pl.ds(0, CHUNK), :],
                                      bufs[ob], sems[ob]).wait()

            pltpu.async_copy(pre_hbm.at[cidx[ob]], bufs[ob], semg[ob])

        pltpu.make_async_copy(pre_hbm.at[cidx[b]], buf, semg[b]).wait()

        for q in range(CHUNK // L):
            vals16 = vals[b][pl.ds(q * L, L)]
            for t in range(L):
                e = q * L + t
                bval = _bcast_lane(vals16, t)
                for j in range(D // L):
                    sl = pl.ds(j * L, L)
                    buf[e, sl] = buf[e, sl] * bval

        for q in range(CHUNK // L):
            sl = pl.ds(q * L, L)
            sidx[b][sl] = ridx[b][sl]

        pltpu.async_copy(buf, acc_sh.at[sidx[b]], sems[b], add=True)

        @pl.when(c + 2 < NCHUNK)
        def _prefetch_idx():
            for src, dst in idx_copies(c + 2, b):
                pltpu.async_copy(src, dst, semi[b])

    def group(g, _):
        slot(2 * g, 0)
        slot(2 * g + 1, 1)
        return ()

    lax.fori_loop(0, NCHUNK // 2, group, ())
    slot(jnp.int32(NCHUNK - 1), 0)

    for b in range(2):
        pltpu.make_async_copy(pre_hbm.at[pl.ds(0, CHUNK), :],
                              bufs[b], sems[b]).wait()
    plsc.subcore_barrier()

    pltpu.sync_copy(acc_sh.at[pl.ds(arow0, ROWS_PER_TILE), :],
                    out_hbm.at[cid, pl.ds(arow0, ROWS_PER_TILE), :])

    @pl.when(sid == 0)
    def _copy_tail():
        pltpu.sync_copy(acc_sh.at[pl.ds(TAIL_ROW0, TAIL), :],
                        out_hbm.at[cid, pl.ds(TAIL_ROW0, TAIL), :])


@jax.jit
def _sc_spmm(rows, cols, vals, pre):
    mesh = plsc.VectorSubcoreMesh(core_axis_name="c", subcore_axis_name="s",
                                  num_cores=NC, num_subcores=NS)
    return pl.kernel(
        _sc_spmm_body,
        out_type=jax.ShapeDtypeStruct((NC, N, D), jnp.float32),
        mesh=mesh,
        scratch_types=[
            pltpu.VMEM((CHUNK,), jnp.int32),
            pltpu.VMEM((CHUNK,), jnp.int32),
            pltpu.VMEM((CHUNK,), jnp.int32),
            pltpu.VMEM((CHUNK,), jnp.int32),
            pltpu.VMEM((CHUNK,), jnp.int32),
            pltpu.VMEM((CHUNK,), jnp.int32),
            pltpu.VMEM((CHUNK,), jnp.float32),
            pltpu.VMEM((CHUNK,), jnp.float32),
            pltpu.VMEM((CHUNK, D), jnp.float32),
            pltpu.VMEM((CHUNK, D), jnp.float32),
            pltpu.VMEM_SHARED((N, D), jnp.float32),
            pltpu.SemaphoreType.DMA,
            pltpu.SemaphoreType.DMA,
            pltpu.SemaphoreType.DMA,
            pltpu.SemaphoreType.DMA,
            pltpu.SemaphoreType.DMA,
            pltpu.SemaphoreType.DMA,
        ],
    )(rows, cols, vals, pre)


def _tc_matmul_body(a_ref, pre_all_ref, agg_ref):
    agg_ref[...] = jnp.dot(a_ref[...], pre_all_ref[...],
                           preferred_element_type=jnp.float32)


@jax.jit
def _tc_matmul(A, pre):
    return pl.pallas_call(
        _tc_matmul_body,
        grid=(N // BM,),
        in_specs=[
            pl.BlockSpec((BM, N), lambda i: (i, 0)),
            pl.BlockSpec((N, D), lambda i: (0, 0)),
        ],
        out_specs=pl.BlockSpec((BM, D), lambda i: (i, 0)),
        out_shape=jax.ShapeDtypeStruct((N, D), jnp.float32),
    )(A, pre)


def _tc_epilogue_body_first(pre_ref, agg_ref, part_ref, w2_ref, b2_ref,
                            w3_ref, b3_ref, emb_ref, fin_ref):
    pre_i = pre_ref[...]
    all_emb = pre_i + agg_ref[...]
    sub_emb = pre_i * (part_ref[0] + part_ref[1])
    lin2 = jnp.dot(all_emb, w2_ref[...],
                   preferred_element_type=jnp.float32) + b2_ref[...]
    lin3 = jnp.dot(sub_emb, w3_ref[...],
                   preferred_element_type=jnp.float32) + b3_ref[...]
    emb = jax.nn.sigmoid(lin2) + jnp.where(lin3 > 0, lin3, NEG_SLOPE * lin3)
    emb_ref[...] = emb
    nrm = jnp.sqrt(jnp.sum(emb * emb, axis=1, keepdims=True))
    norm = emb / jnp.maximum(nrm, 1e-12)
    fin_ref[...] = jnp.concatenate([2.0 * pre_i, norm], axis=1)


def _tc_epilogue_body_second(pre_ref, agg_ref, part_ref, w2_ref, b2_ref,
                             w3_ref, b3_ref, fin_prev_ref, emb_ref, fin_ref):
    del fin_prev_ref
    pre_i = pre_ref[...]
    all_emb = pre_i + agg_ref[...]
    sub_emb = pre_i * (part_ref[0] + part_ref[1])
    lin2 = jnp.dot(all_emb, w2_ref[...],
                   preferred_element_type=jnp.float32) + b2_ref[...]
    lin3 = jnp.dot(sub_emb, w3_ref[...],
                   preferred_element_type=jnp.float32) + b3_ref[...]
    emb = jax.nn.sigmoid(lin2) + jnp.where(lin3 > 0, lin3, NEG_SLOPE * lin3)
    emb_ref[...] = emb
    nrm = jnp.sqrt(jnp.sum(emb * emb, axis=1, keepdims=True))
    fin_ref[...] = emb / jnp.maximum(nrm, 1e-12)


_EPI_IN_SPECS = [
    pl.BlockSpec((BM, D), lambda i: (i, 0)),
    pl.BlockSpec((BM, D), lambda i: (i, 0)),
    pl.BlockSpec((NC, BM, D), lambda i: (0, i, 0)),
    pl.BlockSpec((D, D), lambda i: (0, 0)),
    pl.BlockSpec((1, D), lambda i: (0, 0)),
    pl.BlockSpec((D, D), lambda i: (0, 0)),
    pl.BlockSpec((1, D), lambda i: (0, 0)),
]


@jax.jit
def _tc_epilogue_first(pre, agg, parts, W2, b2, W3, b3):
    return pl.pallas_call(
        _tc_epilogue_body_first,
        grid=(N // BM,),
        in_specs=_EPI_IN_SPECS,
        out_specs=[
            pl.BlockSpec((BM, D), lambda i: (i, 0)),
            pl.BlockSpec((BM, 2 * D), lambda i: (i, 0)),
        ],
        out_shape=[
            jax.ShapeDtypeStruct((N, D), jnp.float32),
            jax.ShapeDtypeStruct((N, 3 * D), jnp.float32),
        ],
    )(pre, agg, parts, W2, b2.reshape(1, D), W3, b3.reshape(1, D))


@functools.partial(jax.jit, donate_argnums=(7,))
def _tc_epilogue_second(pre, agg, parts, W2, b2, W3, b3, fin_prev):
    return pl.pallas_call(
        _tc_epilogue_body_second,
        grid=(N // BM,),
        in_specs=_EPI_IN_SPECS + [pl.BlockSpec(memory_space=pl.ANY)],
        out_specs=[
            pl.BlockSpec((BM, D), lambda i: (i, 0)),
            pl.BlockSpec((BM, D), lambda i: (i, 2)),
        ],
        out_shape=[
            jax.ShapeDtypeStruct((N, D), jnp.float32),
            jax.ShapeDtypeStruct((N, 3 * D), jnp.float32),
        ],
        input_output_aliases={7: 1},
    )(pre, agg, parts, W2, b2.reshape(1, D), W3, b3.reshape(1, D), fin_prev)


def kernel(A, sub_indices, sub_values, X,
           W2_0, b2_0, W3_0, b3_0,
           W2_1, b2_1, W3_1, b3_1):
    rows = sub_indices[0].astype(jnp.int32)
    cols = sub_indices[1].astype(jnp.int32)
    parts = _sc_spmm(rows, cols, sub_values, X)
    agg = _tc_matmul(A, X)
    emb, fin = _tc_epilogue_first(X, agg, parts, W2_0, b2_0, W3_0, b3_0)
    parts = _sc_spmm(rows, cols, sub_values, emb)
    agg = _tc_matmul(A, emb)
    _, fin = _tc_epilogue_second(emb, agg, parts, W2_1, b2_1, W3_1, b3_1, fin)
    return fin
